# baseline (speedup 1.0000x reference)
import functools

import jax
import jax.numpy as jnp
from jax import lax
from jax.experimental import pallas as pl
from jax.experimental.pallas import tpu as pltpu
from jax.experimental.pallas import tpu_sc as plsc

NUM_SERVICES = 100000
ENC = 512
EMB = 64
BATCH = 16384
HALF = BATCH // 2

NC = 2
NS = 16
NW = NC * NS
B_PER_W = HALF // NW
CHUNK = 64
N_CHUNKS = B_PER_W // CHUNK
N_PAIRS = N_CHUNKS // 2


def _make_sc_gather():
    mesh = plsc.VectorSubcoreMesh(core_axis_name="c", subcore_axis_name="s")

    @functools.partial(
        pl.kernel,
        mesh=mesh,
        out_type=jax.ShapeDtypeStruct((HALF, ENC), jnp.float32),
        scratch_types=[
            pltpu.VMEM((B_PER_W,), jnp.int32),
            pltpu.VMEM((CHUNK, ENC), jnp.float32),
            pltpu.VMEM((CHUNK, ENC), jnp.float32),
            pltpu.SemaphoreType.DMA,
            pltpu.SemaphoreType.DMA,
            pltpu.SemaphoreType.DMA,
            pltpu.SemaphoreType.DMA,
        ],
    )
    def gather_k(idx_hbm, table_hbm, out_hbm,
                 idx_v, rows0, rows1, g0, g1, o0, o1):
        wid = lax.axis_index("s") * NC + lax.axis_index("c")
        base = wid * B_PER_W
        pltpu.sync_copy(idx_hbm.at[pl.ds(base, B_PER_W)], idx_v)
        rows = (rows0, rows1)
        gsems = (g0, g1)
        osems = (o0, o1)

        def start_gather(ci, b):
            pltpu.async_copy(
                table_hbm.at[idx_v.at[pl.ds(ci * CHUNK, CHUNK)]],
                rows[b], gsems[b])

        def wait_gather(b):
            pltpu.make_async_copy(
                table_hbm.at[pl.ds(0, CHUNK)], rows[b], gsems[b]).wait()

        def wait_out(b):
            pltpu.make_async_copy(
                table_hbm.at[pl.ds(0, CHUNK)], rows[b], osems[b]).wait()

        start_gather(0, 0)
        start_gather(1, 1)
        for ci in range(N_CHUNKS):
            b = ci % 2
            wait_gather(b)
            if ci >= 2:
                pass
            pltpu.async_copy(
                rows[b], out_hbm.at[pl.ds(base + ci * CHUNK, CHUNK)], osems[b])
            if ci + 2 < N_CHUNKS:
                wait_out(b)
                start_gather(ci + 2, b)
        for ci in (N_CHUNKS - 2, N_CHUNKS - 1):
            wait_out(ci % 2)

    return gather_k


_sc_gather = _make_sc_gather()


def _mm_body(s_ref, e_ref, o_ref):
    o_ref[...] = jnp.dot(s_ref[...], e_ref[...], preferred_element_type=jnp.float32)


def _tc_matmul(gathered, emb):
    return pl.pallas_call(
        _mm_body,
        grid=(4,),
        in_specs=[
            pl.BlockSpec((HALF // 4, ENC), lambda i: (i, 0)),
            pl.BlockSpec((ENC, EMB), lambda i: (0, 0)),
        ],
        out_specs=pl.BlockSpec((HALF // 4, EMB), lambda i: (i, 0)),
        out_shape=jax.ShapeDtypeStruct((HALF, EMB), jnp.float32),
    )(gathered, emb)


def kernel(data, service_matrix, embedding_matrix):
    g0 = _sc_gather(data[:HALF], service_matrix)
    g1 = _sc_gather(data[HALF:], service_matrix)
    o0 = _tc_matmul(g0, embedding_matrix)
    o1 = _tc_matmul(g1, embedding_matrix)
    return jnp.concatenate([o0, o1], axis=0)


# speedup vs baseline: 1.1357x; 1.1357x over previous
import functools

import jax
import jax.numpy as jnp
from jax import lax
from jax.experimental import pallas as pl
from jax.experimental.pallas import tpu as pltpu
from jax.experimental.pallas import tpu_sc as plsc

NUM_SERVICES = 100000
ENC = 512
EMB = 64
BATCH = 16384

NC = 2
NS = 16
NW = NC * NS
B_PER_W = BATCH // NW
CHUNK = 64
N_CHUNKS = B_PER_W // CHUNK
NBUF = 3


def _make_sc_gather():
    mesh = plsc.VectorSubcoreMesh(core_axis_name="c", subcore_axis_name="s")

    @functools.partial(
        pl.kernel,
        mesh=mesh,
        out_type=jax.ShapeDtypeStruct((BATCH, ENC), jnp.float32),
        scratch_types=[
            pltpu.VMEM((B_PER_W,), jnp.int32),
            pltpu.VMEM((CHUNK, ENC), jnp.float32),
            pltpu.VMEM((CHUNK, ENC), jnp.float32),
            pltpu.VMEM((CHUNK, ENC), jnp.float32),
            pltpu.SemaphoreType.DMA((NBUF,)),
            pltpu.SemaphoreType.DMA((NBUF,)),
        ],
    )
    def gather_k(idx_hbm, table_hbm, out_hbm,
                 idx_v, rows0, rows1, rows2, gsem, osem):
        wid = lax.axis_index("s") * NC + lax.axis_index("c")
        base = wid * B_PER_W
        pltpu.sync_copy(idx_hbm.at[pl.ds(base, B_PER_W)], idx_v)
        rows = (rows0, rows1, rows2)

        def start_gather(ci, b):
            pltpu.async_copy(
                table_hbm.at[idx_v.at[pl.ds(ci * CHUNK, CHUNK)]],
                rows[b], gsem.at[b])

        def wait_gather(b):
            pltpu.make_async_copy(
                table_hbm.at[pl.ds(0, CHUNK)], rows[b], gsem.at[b]).wait()

        def wait_out(b):
            pltpu.make_async_copy(
                table_hbm.at[pl.ds(0, CHUNK)], rows[b], osem.at[b]).wait()

        for j in range(NBUF):
            start_gather(j, j)
        for ci in range(N_CHUNKS):
            b = ci % NBUF
            wait_gather(b)
            pltpu.async_copy(
                rows[b], out_hbm.at[pl.ds(base + ci * CHUNK, CHUNK)],
                osem.at[b])
            if ci + NBUF < N_CHUNKS:
                wait_out(b)
                start_gather(ci + NBUF, b)
        for ci in range(N_CHUNKS - NBUF, N_CHUNKS):
            wait_out(ci % NBUF)

    return gather_k


_sc_gather = _make_sc_gather()


def _mm_body(s_ref, e_ref, o_ref):
    o_ref[...] = jnp.dot(s_ref[...], e_ref[...], preferred_element_type=jnp.float32)


def kernel(data, service_matrix, embedding_matrix):
    gathered = _sc_gather(data, service_matrix)
    out = pl.pallas_call(
        _mm_body,
        grid=(8,),
        in_specs=[
            pl.BlockSpec((BATCH // 8, ENC), lambda i: (i, 0)),
            pl.BlockSpec((ENC, EMB), lambda i: (0, 0)),
        ],
        out_specs=pl.BlockSpec((BATCH // 8, EMB), lambda i: (i, 0)),
        out_shape=jax.ShapeDtypeStruct((BATCH, EMB), jnp.float32),
    )(gathered, embedding_matrix)
    return out
